# baseline (speedup 1.0000x reference)
import functools

import jax
import jax.numpy as jnp
from jax import lax
from jax.experimental import pallas as pl
from jax.experimental.pallas import tpu as pltpu, tpu_sc as plsc

NC = 2
NS = 16
L = 16
NW = NC * NS


def _gather_add_kernel(B, D):
    b_per_w = B // NW
    CH = 16
    n_chunks = b_per_w // CH
    NR = 4
    NX = 2
    vecs_per_row = D // L

    mesh = plsc.VectorSubcoreMesh(core_axis_name="c", subcore_axis_name="s")

    @functools.partial(
        pl.kernel,
        out_type=jax.ShapeDtypeStruct((B, D), jnp.float32),
        mesh=mesh,
        scratch_types=[
            pltpu.VMEM((n_chunks, CH), jnp.int32),
            [pltpu.VMEM((CH, D), jnp.float32) for _ in range(NR)],
            [pltpu.VMEM((CH, D), jnp.float32) for _ in range(NX)],
            [pltpu.SemaphoreType.DMA for _ in range(NR)],
            [pltpu.SemaphoreType.DMA for _ in range(NX)],
            [pltpu.SemaphoreType.DMA for _ in range(NR)],
        ],
    )
    def body(x_hbm, emb_hbm, idx_hbm, out_hbm, idx_v, rows, xs, gsem, xsem, ssem):
        wid = lax.axis_index("s") * NC + lax.axis_index("c")
        base = wid * b_per_w
        pltpu.sync_copy(idx_hbm.at[wid], idx_v)

        def issue_gather(g):
            q = g % NR
            return pltpu.async_copy(emb_hbm.at[idx_v.at[g]], rows[q], gsem[q])

        def issue_x(c):
            p = c % NX
            return pltpu.async_copy(
                x_hbm.at[pl.ds(base + c * CH, CH), :], xs[p], xsem[p])

        def issue_store(c):
            q = c % NR
            return pltpu.async_copy(
                rows[q], out_hbm.at[pl.ds(base + c * CH, CH), :], ssem[q])

        g_h = [None] * NR
        x_h = [None] * NX
        s_h = [None] * NR
        for g in range(min(2, n_chunks)):
            g_h[g % NR] = issue_gather(g)
        for c in range(min(2, n_chunks)):
            x_h[c % NX] = issue_x(c)

        for c in range(n_chunks):
            g = c + 2
            if g < n_chunks:
                q = g % NR
                if s_h[q] is not None:
                    s_h[q].wait()
                    s_h[q] = None
                g_h[q] = issue_gather(g)

            q, p = c % NR, c % NX
            g_h[q].wait()
            x_h[p].wait()

            @plsc.parallel_loop(0, CH, 1)
            def row(r):
                for j in range(vecs_per_row):
                    sl = pl.ds(j * L, L)
                    rows[q][r, sl] = rows[q][r, sl] + xs[p][r, sl]

            if c + 2 < n_chunks:
                x_h[p] = issue_x(c + 2)
            s_h[q] = issue_store(c)

        for q in range(NR):
            if s_h[q] is not None:
                s_h[q].wait()

    return body


@jax.jit
def _run(x_tc, embeddings_tc, times_t):
    B, D = x_tc.shape
    fn = _gather_add_kernel(B, D)
    b_per_w = B // NW
    CH = 16
    idx = times_t.astype(jnp.int32).reshape(NW, b_per_w // CH, CH)
    return fn(x_tc, embeddings_tc, idx)


def kernel(x_tc, embeddings_tc, offset, times_t):
    if times_t is None:
        times_t = offset + jnp.arange(x_tc.shape[0], dtype=jnp.int32)
    return _run(x_tc, embeddings_tc, times_t)


# speedup vs baseline: 1.0352x; 1.0352x over previous
import functools

import jax
import jax.numpy as jnp
from jax import lax
from jax.experimental import pallas as pl
from jax.experimental.pallas import tpu as pltpu, tpu_sc as plsc

NC = 2
NS = 16
L = 16
NW = NC * NS


def _gather_add_kernel(B, D):
    b_per_w = B // NW
    CH = 16
    n_chunks = b_per_w // CH
    NR = 4
    NX = 3
    vecs_per_row = D // L

    mesh = plsc.VectorSubcoreMesh(core_axis_name="c", subcore_axis_name="s")

    @functools.partial(
        pl.kernel,
        out_type=jax.ShapeDtypeStruct((B, D), jnp.float32),
        mesh=mesh,
        scratch_types=[
            pltpu.VMEM((n_chunks, CH), jnp.int32),
            [pltpu.VMEM((CH, D), jnp.float32) for _ in range(NR)],
            [pltpu.VMEM((CH, D), jnp.float32) for _ in range(NX)],
            [pltpu.SemaphoreType.DMA for _ in range(NR)],
            [pltpu.SemaphoreType.DMA for _ in range(NX)],
            [pltpu.SemaphoreType.DMA for _ in range(NR)],
        ],
    )
    def body(x_hbm, emb_hbm, idx_hbm, out_hbm, idx_v, rows, xs, gsem, xsem, ssem):
        wid = lax.axis_index("s") * NC + lax.axis_index("c")
        base = wid * b_per_w
        pltpu.sync_copy(idx_hbm.at[wid], idx_v)

        def issue_gather(g):
            q = g % NR
            return pltpu.async_copy(emb_hbm.at[idx_v.at[g]], rows[q], gsem[q])

        def issue_x(c):
            p = c % NX
            return pltpu.async_copy(
                x_hbm.at[pl.ds(base + c * CH, CH), :], xs[p], xsem[p])

        def issue_store(c):
            q = c % NR
            return pltpu.async_copy(
                rows[q], out_hbm.at[pl.ds(base + c * CH, CH), :], ssem[q])

        g_h = [None] * NR
        x_h = [None] * NX
        s_h = [None] * NR
        for g in range(min(2, n_chunks)):
            g_h[g % NR] = issue_gather(g)
        for c in range(min(NX, n_chunks)):
            x_h[c % NX] = issue_x(c)

        for c in range(n_chunks):
            g = c + 2
            if g < n_chunks:
                q = g % NR
                if s_h[q] is not None:
                    s_h[q].wait()
                    s_h[q] = None
                g_h[q] = issue_gather(g)

            q, p = c % NR, c % NX
            g_h[q].wait()
            x_h[p].wait()

            def row(r, _):
                for j in range(vecs_per_row):
                    sl = pl.ds(j * L, L)
                    rows[q][r, sl] = rows[q][r, sl] + xs[p][r, sl]
                return 0

            lax.fori_loop(0, CH, row, 0)

            if c + NX < n_chunks:
                x_h[p] = issue_x(c + NX)
            s_h[q] = issue_store(c)

        for q in range(NR):
            if s_h[q] is not None:
                s_h[q].wait()

    return body


@jax.jit
def _run(x_tc, embeddings_tc, times_t):
    B, D = x_tc.shape
    fn = _gather_add_kernel(B, D)
    b_per_w = B // NW
    CH = 16
    idx = times_t.astype(jnp.int32).reshape(NW, b_per_w // CH, CH)
    return fn(x_tc, embeddings_tc, idx)


def kernel(x_tc, embeddings_tc, offset, times_t):
    if times_t is None:
        times_t = offset + jnp.arange(x_tc.shape[0], dtype=jnp.int32)
    return _run(x_tc, embeddings_tc, times_t)


# speedup vs baseline: 1.0434x; 1.0079x over previous
import functools

import jax
import jax.numpy as jnp
from jax import lax
from jax.experimental import pallas as pl
from jax.experimental.pallas import tpu as pltpu, tpu_sc as plsc

NC = 2
NS = 16
L = 16
NW = NC * NS


def _gather_add_kernel(B, D):
    b_per_w = B // NW
    CH = 16
    n_chunks = b_per_w // CH
    NR = 4
    NX = 3
    vecs_per_row = D // L

    mesh = plsc.VectorSubcoreMesh(core_axis_name="c", subcore_axis_name="s")

    @functools.partial(
        pl.kernel,
        out_type=jax.ShapeDtypeStruct((B, D), jnp.float32),
        mesh=mesh,
        scratch_types=[
            pltpu.VMEM((b_per_w,), jnp.int32),
            [pltpu.VMEM((CH, D), jnp.float32) for _ in range(NR)],
            [pltpu.VMEM((CH, D), jnp.float32) for _ in range(NX)],
            [pltpu.SemaphoreType.DMA for _ in range(NR)],
            [pltpu.SemaphoreType.DMA for _ in range(NX)],
            [pltpu.SemaphoreType.DMA for _ in range(NR)],
        ],
    )
    def body(x_hbm, emb_hbm, idx_hbm, out_hbm, idx_v, rows, xs, gsem, xsem, ssem):
        wid = lax.axis_index("s") * NC + lax.axis_index("c")
        base = wid * b_per_w
        pltpu.sync_copy(idx_hbm.at[pl.ds(base, b_per_w)], idx_v)

        def issue_gather(g):
            q = g % NR
            return pltpu.async_copy(
                emb_hbm.at[idx_v.at[pl.ds(g * CH, CH)]], rows[q], gsem[q])

        def issue_x(c):
            p = c % NX
            return pltpu.async_copy(
                x_hbm.at[pl.ds(base + c * CH, CH), :], xs[p], xsem[p])

        def issue_store(c):
            q = c % NR
            return pltpu.async_copy(
                rows[q], out_hbm.at[pl.ds(base + c * CH, CH), :], ssem[q])

        g_h = [None] * NR
        x_h = [None] * NX
        s_h = [None] * NR
        for g in range(min(2, n_chunks)):
            g_h[g % NR] = issue_gather(g)
        for c in range(min(NX, n_chunks)):
            x_h[c % NX] = issue_x(c)

        for c in range(n_chunks):
            g = c + 2
            if g < n_chunks:
                q = g % NR
                if s_h[q] is not None:
                    s_h[q].wait()
                    s_h[q] = None
                g_h[q] = issue_gather(g)

            q, p = c % NR, c % NX
            g_h[q].wait()
            x_h[p].wait()

            def row(r, _):
                for j in range(vecs_per_row):
                    sl = pl.ds(j * L, L)
                    rows[q][r, sl] = rows[q][r, sl] + xs[p][r, sl]
                return 0

            lax.fori_loop(0, CH, row, 0)

            if c + NX < n_chunks:
                x_h[p] = issue_x(c + NX)
            s_h[q] = issue_store(c)

        for q in range(NR):
            if s_h[q] is not None:
                s_h[q].wait()

    return body


@jax.jit
def _run(x_tc, embeddings_tc, times_t):
    B, D = x_tc.shape
    fn = _gather_add_kernel(B, D)
    return fn(x_tc, embeddings_tc, times_t.astype(jnp.int32))


def kernel(x_tc, embeddings_tc, offset, times_t):
    if times_t is None:
        times_t = offset + jnp.arange(x_tc.shape[0], dtype=jnp.int32)
    return _run(x_tc, embeddings_tc, times_t)
